# baseline (speedup 1.0000x reference)
import functools

import jax
import jax.numpy as jnp
import numpy as np
from jax import lax
from jax.experimental import pallas as pl
from jax.experimental.pallas import tpu as pltpu
from jax.experimental.pallas import tpu_sc as plsc

NPAD = 10240
DIM = 128
NC, NS = 2, 16
NW = NC * NS
CH = 128


SUP = 16


def _pipe_superchunk(table_hbm, idxg, idxs, acc, rows0, rows1, gs0, gs1,
                     ss0, ss1):
    def gather(j, rows, sem):
        pltpu.async_copy(table_hbm.at[idxg.at[j]], rows, sem)

    def gather_wait(j, rows, sem):
        pltpu.make_async_copy(table_hbm.at[idxg.at[j]], rows, sem).wait()

    def scat(j, rows, sem):
        pltpu.async_copy(rows, acc.at[idxs.at[j]], sem, add=True)

    def scat_wait(j, rows, sem):
        pltpu.make_async_copy(rows, acc.at[idxs.at[j]], sem).wait()

    gather(0, rows0, gs0)
    gather(1, rows1, gs1)
    for i in range(SUP // 2 - 1):
        j = 2 * i
        gather_wait(j, rows0, gs0)
        scat(j, rows0, ss0)
        gather_wait(j + 1, rows1, gs1)
        scat(j + 1, rows1, ss1)
        scat_wait(j, rows0, ss0)
        gather(j + 2, rows0, gs0)
        scat_wait(j + 1, rows1, ss1)
        gather(j + 3, rows1, gs1)
    jl = SUP - 2
    gather_wait(jl, rows0, gs0)
    scat(jl, rows0, ss0)
    gather_wait(jl + 1, rows1, gs1)
    scat(jl + 1, rows1, ss1)
    scat_wait(jl, rows0, ss0)
    scat_wait(jl + 1, rows1, ss1)


def _sc_msg_body(nchunk, hw_hbm, rowi_hbm, coli_hbm, zeros_hbm, out_hbm,
                 acc, rba, cba, rows0, rows1, gs0, gs1, ss0, ss1):
    c = lax.axis_index("c")
    s = lax.axis_index("s")
    rpt = NPAD // NS
    nsup_all = 2 * (nchunk // SUP)

    @pl.when(c == 0)
    def _():
        pltpu.sync_copy(zeros_hbm.at[pl.ds(s * rpt, rpt)],
                        acc.at[pl.ds(s * rpt, rpt)])
        plsc.subcore_barrier()

        @pl.loop(0, nsup_all)
        def _(t):
            base = (s * nsup_all + t) * SUP
            pltpu.sync_copy(rowi_hbm.at[pl.ds(base, SUP)], rba)
            pltpu.sync_copy(coli_hbm.at[pl.ds(base, SUP)], cba)
            _pipe_superchunk(hw_hbm, rba, cba, acc, rows0, rows1, gs0, gs1,
                             ss0, ss1)

        plsc.subcore_barrier()
        pltpu.sync_copy(acc.at[pl.ds(s * rpt, rpt)],
                        out_hbm.at[pl.ds(s * rpt, rpt)])


@functools.lru_cache(maxsize=None)
def _make_sc_msg(nchunk):
    mesh = plsc.VectorSubcoreMesh(core_axis_name="c", subcore_axis_name="s")
    return pl.kernel(
        functools.partial(_sc_msg_body, nchunk),
        out_type=jax.ShapeDtypeStruct((NPAD, DIM), jnp.float32),
        mesh=mesh,
        scratch_types=[
            pltpu.VMEM_SHARED((NPAD, DIM), jnp.float32),
            pltpu.VMEM((SUP, CH), jnp.int32),
            pltpu.VMEM((SUP, CH), jnp.int32),
            pltpu.VMEM((CH, DIM), jnp.float32),
            pltpu.VMEM((CH, DIM), jnp.float32),
            pltpu.SemaphoreType.DMA,
            pltpu.SemaphoreType.DMA,
            pltpu.SemaphoreType.DMA,
            pltpu.SemaphoreType.DMA,
        ],
    )


def _sc_cnt_body(nchunk, ai_hbm, bi_hbm, coli_hbm, zeros_hbm,
                 out_hbm, acc, aba, bba, cba, rows0, rows1, ss0, ss1):
    c = lax.axis_index("c")
    s = lax.axis_index("s")
    wid = s * NC + c
    rpt = NPAD // NS
    pltpu.sync_copy(zeros_hbm.at[pl.ds(s * rpt, rpt)],
                    acc.at[pl.ds(s * rpt, rpt)])
    pltpu.sync_copy(zeros_hbm.at[pl.ds(0, CH)], rows0)
    pltpu.sync_copy(zeros_hbm.at[pl.ds(0, CH)], rows1)
    plsc.subcore_barrier()

    ones16 = jnp.full((16,), 1.0, jnp.float32)
    zeros16 = jnp.zeros((16,), jnp.float32)
    lane = lax.iota(jnp.int32, 16)
    rows_bufs = (rows0, rows1)
    sems = (ss0, ss1)

    def set_vals(j, rows, val):
        for g in range(CH // 16):
            kvec = lane + (g * 16)
            sl = pl.ds(g * 16, 16)
            plsc.store_scatter(rows, [kvec, aba[j, sl]], val)
            plsc.store_scatter(rows, [kvec, bba[j, sl] + 8], val)

    def scat(j, rows, sem):
        pltpu.async_copy(rows, acc.at[cba.at[j]], sem, add=True)

    def scat_wait(j, rows, sem):
        pltpu.make_async_copy(rows, acc.at[cba.at[j]], sem).wait()

    @pl.loop(0, nchunk // SUP)
    def _(t):
        base = wid * nchunk + t * SUP
        pltpu.sync_copy(ai_hbm.at[pl.ds(base, SUP)], aba)
        pltpu.sync_copy(bi_hbm.at[pl.ds(base, SUP)], bba)
        pltpu.sync_copy(coli_hbm.at[pl.ds(base, SUP)], cba)
        for j in range(SUP):
            sl = j % 2
            set_vals(j, rows_bufs[sl], ones16)
            scat(j, rows_bufs[sl], sems[sl])
            if j >= 1:
                psl = (j - 1) % 2
                scat_wait(j - 1, rows_bufs[psl], sems[psl])
                set_vals(j - 1, rows_bufs[psl], zeros16)
        scat_wait(SUP - 1, rows_bufs[(SUP - 1) % 2], sems[(SUP - 1) % 2])
        set_vals(SUP - 1, rows_bufs[(SUP - 1) % 2], zeros16)

    plsc.subcore_barrier()
    pltpu.sync_copy(acc.at[pl.ds(s * rpt, rpt)],
                    out_hbm.at[c, pl.ds(s * rpt, rpt)])


@functools.lru_cache(maxsize=None)
def _make_sc_cnt(nchunk):
    mesh = plsc.VectorSubcoreMesh(core_axis_name="c", subcore_axis_name="s")
    return pl.kernel(
        functools.partial(_sc_cnt_body, nchunk),
        out_type=jax.ShapeDtypeStruct((NC, NPAD, DIM), jnp.float32),
        mesh=mesh,
        scratch_types=[
            pltpu.VMEM_SHARED((NPAD, DIM), jnp.float32),
            pltpu.VMEM((SUP, CH), jnp.int32),
            pltpu.VMEM((SUP, CH), jnp.int32),
            pltpu.VMEM((SUP, CH), jnp.int32),
            pltpu.VMEM((CH, DIM), jnp.float32),
            pltpu.VMEM((CH, DIM), jnp.float32),
            pltpu.SemaphoreType.DMA,
            pltpu.SemaphoreType.DMA,
        ],
        compiler_params=pltpu.CompilerParams(needs_layout_passes=False),
    )


def _prep_body(x0_ref, x1_ref, e1_ref, e2_ref, w0_ref, hw0_ref):
    oh0 = (x0_ref[:] == lax.broadcasted_iota(jnp.int32, (NPAD, 8), 1)
           ).astype(jnp.float32)
    oh1 = (x1_ref[:] == lax.broadcasted_iota(jnp.int32, (NPAD, 8), 1)
           ).astype(jnp.float32)
    h0 = (jnp.dot(oh0, e1_ref[:], preferred_element_type=jnp.float32)
          + jnp.dot(oh1, e2_ref[:], preferred_element_type=jnp.float32))
    hw0_ref[:] = jnp.dot(h0, w0_ref[:], preferred_element_type=jnp.float32)


def _layer_body(nreal, relu, has_w, p_ref, hw_ref, cntp_ref, v_ref, g_ref,
                b_ref, wn_ref, out_ref):
    cnt = cntp_ref[0] + cntp_ref[1]
    scol = jnp.sum(cnt * v_ref[:], axis=1, keepdims=True)
    out = p_ref[:] + hw_ref[:] + scol
    rmask = lax.broadcasted_iota(jnp.int32, (NPAD, 1), 0) < nreal
    outm = jnp.where(rmask, out, 0.0)
    mean = jnp.sum(outm, axis=0, keepdims=True) * (1.0 / nreal)
    d = jnp.where(rmask, out - mean, 0.0)
    var = jnp.sum(d * d, axis=0, keepdims=True) * (1.0 / nreal)
    xn = (out - mean) * lax.rsqrt(var + 1e-5) * g_ref[:] + b_ref[:]
    if relu:
        xn = jnp.maximum(xn, 0.0)
    if has_w:
        out_ref[:] = jnp.dot(xn, wn_ref[:], preferred_element_type=jnp.float32)
    else:
        out_ref[:] = xn


def _pool_body(ngroups, h_ref, bt_ref, flw_ref, flb_ref, p1w_ref, p1b_ref,
               p2w_ref, p2b_ref, hg_ref, pred_ref):
    oht = (bt_ref[:] == lax.broadcasted_iota(jnp.int32, (ngroups, NPAD), 0)
           ).astype(jnp.float32)
    pooled = jnp.dot(oht, h_ref[:], preferred_element_type=jnp.float32)
    cnts = jnp.sum(oht, axis=1, keepdims=True)
    pooled = pooled / jnp.maximum(cnts, 1.0)
    hg = jnp.dot(pooled, flw_ref[:], preferred_element_type=jnp.float32) \
        + flb_ref[:]
    hg_ref[:] = hg
    t = jnp.dot(hg, p1w_ref[:], preferred_element_type=jnp.float32) \
        + p1b_ref[:]
    sp = jnp.maximum(t, 0.0) + jnp.log(1.0 + jnp.exp(-jnp.abs(t)))
    pred_ref[:] = jnp.dot(sp, p2w_ref[:], preferred_element_type=jnp.float32) \
        + p2b_ref[:]


def kernel(x, edge_index, edge_attr, batch, x_emb1, x_emb2, Ws, bs, ee1, ee2,
           gammas, betas, flW, flb, p1W, p1b, p2W, p2b):
    n = x.shape[0]
    e = edge_index.shape[1]
    nlayers = Ws.shape[0]
    feat = flW.shape[1]
    ngroups = 64

    nchunk = -(-e // (NW * CH))
    nchunk = -(-nchunk // SUP) * SUP
    epw = nchunk * CH
    e_pad = NW * epw

    f32 = jnp.float32
    i32 = jnp.int32

    x0 = jnp.pad(x[:, 0:1], ((0, NPAD - n), (0, 0))).astype(i32)
    x1 = jnp.pad(x[:, 1:2], ((0, NPAD - n), (0, 0))).astype(i32)
    e1 = x_emb1[:8].astype(f32)
    e2 = jnp.pad(x_emb2, ((0, 5), (0, 0))).astype(f32)
    rowi = jnp.pad(edge_index[0], (0, e_pad - e),
                   constant_values=NPAD - 1).astype(i32).reshape(-1, CH)
    coli = jnp.pad(edge_index[1], (0, e_pad - e),
                   constant_values=NPAD - 1).astype(i32).reshape(-1, CH)
    ai = jnp.pad(edge_attr[:, 0], (0, e_pad - e),
                 constant_values=3).astype(i32).reshape(-1, CH)
    bi = jnp.pad(edge_attr[:, 1], (0, e_pad - e),
                 constant_values=3).astype(i32).reshape(-1, CH)
    bt = jnp.pad(batch, (0, NPAD - n),
                 constant_values=ngroups).reshape(1, NPAD).astype(i32)
    zeros_nk = jnp.zeros((NPAD, DIM), f32)
    p2w_pad = jnp.pad(p2W, ((0, 0), (0, DIM - p2W.shape[1]))).astype(f32)
    p2b_pad = jnp.pad(p2b, (0, DIM - p2b.shape[0])).reshape(1, DIM).astype(f32)

    cntp = _make_sc_cnt(nchunk)(ai, bi, coli, zeros_nk)

    prep = pl.pallas_call(
        _prep_body,
        out_shape=jax.ShapeDtypeStruct((NPAD, DIM), f32),
    )
    hw = prep(x0, x1, e1, e2, Ws[0].astype(f32))

    sc_msg = _make_sc_msg(nchunk)
    for l in range(nlayers):
        partials = sc_msg(hw, rowi, coli, zeros_nk)
        has_w = l < nlayers - 1
        v = jnp.concatenate([
            ee1[l, :3, 0], jnp.zeros((5,), f32),
            ee2[l, :3, 0], jnp.zeros((DIM - 11,), f32)]).reshape(1, DIM)
        g = gammas[l].reshape(1, DIM).astype(f32)
        b = betas[l].reshape(1, DIM).astype(f32)
        wn = (Ws[l + 1] if has_w else jnp.zeros((DIM, DIM))).astype(f32)
        layer = pl.pallas_call(
            functools.partial(_layer_body, n, l < nlayers - 1, has_w),
            out_shape=jax.ShapeDtypeStruct((NPAD, DIM), f32),
        )
        hw = layer(partials, hw, cntp, v, g, b, wn)
    h_final = hw

    pool = pl.pallas_call(
        functools.partial(_pool_body, ngroups),
        out_shape=(
            jax.ShapeDtypeStruct((ngroups, feat), f32),
            jax.ShapeDtypeStruct((ngroups, DIM), f32),
        ),
    )
    hg, pred_pad = pool(h_final, bt, flW.astype(f32),
                        flb.reshape(1, feat).astype(f32), p1W.astype(f32),
                        p1b.reshape(1, p1W.shape[1]).astype(f32), p2w_pad,
                        p2b_pad)
    return (hg, pred_pad[:, :p2W.shape[1]])


# speedup vs baseline: 1.5205x; 1.5205x over previous
import functools

import jax
import jax.numpy as jnp
import numpy as np
from jax import lax
from jax.experimental import pallas as pl
from jax.experimental.pallas import tpu as pltpu
from jax.experimental.pallas import tpu_sc as plsc

NPAD = 10240
DIM = 128
NC, NS = 2, 16
NW = NC * NS
CH = 128


SUP = 16


def _pipe_superchunk(table_hbm, idxg, idxs, acc, rows0, rows1, gs0, gs1,
                     ss0, ss1):
    def gather(j, rows, sem):
        pltpu.async_copy(table_hbm.at[idxg.at[j]], rows, sem)

    def gather_wait(j, rows, sem):
        pltpu.make_async_copy(table_hbm.at[idxg.at[j]], rows, sem).wait()

    def scat(j, rows, sem):
        pltpu.async_copy(rows, acc.at[idxs.at[j]], sem, add=True)

    def scat_wait(j, rows, sem):
        pltpu.make_async_copy(rows, acc.at[idxs.at[j]], sem).wait()

    gather(0, rows0, gs0)
    gather(1, rows1, gs1)
    for i in range(SUP // 2 - 1):
        j = 2 * i
        gather_wait(j, rows0, gs0)
        scat(j, rows0, ss0)
        gather_wait(j + 1, rows1, gs1)
        scat(j + 1, rows1, ss1)
        scat_wait(j, rows0, ss0)
        gather(j + 2, rows0, gs0)
        scat_wait(j + 1, rows1, ss1)
        gather(j + 3, rows1, gs1)
    jl = SUP - 2
    gather_wait(jl, rows0, gs0)
    scat(jl, rows0, ss0)
    gather_wait(jl + 1, rows1, gs1)
    scat(jl + 1, rows1, ss1)
    scat_wait(jl, rows0, ss0)
    scat_wait(jl + 1, rows1, ss1)


def _sc_msg_body(nchunk, t0, hw_hbm, rowi_hbm, coli_hbm, zeros_hbm, out_hbm,
                 acc, rba, cba, rows0, rows1, gs0, gs1, ss0, ss1):
    c = lax.axis_index("c")
    s = lax.axis_index("s")
    rpt = NPAD // NS
    pltpu.sync_copy(zeros_hbm.at[pl.ds(s * rpt, rpt)],
                    acc.at[pl.ds(s * rpt, rpt)])
    plsc.subcore_barrier()

    nsup = nchunk // SUP
    t1 = 2 * nsup - t0
    nsup_me = jnp.where(c == 0, t0, t1)
    base_sup = jnp.where(c == 0, s * t0, NS * t0 + s * t1)

    @pl.loop(0, nsup_me)
    def _(t):
        base = (base_sup + t) * SUP
        pltpu.sync_copy(rowi_hbm.at[pl.ds(base, SUP)], rba)
        pltpu.sync_copy(coli_hbm.at[pl.ds(base, SUP)], cba)
        _pipe_superchunk(hw_hbm, rba, cba, acc, rows0, rows1, gs0, gs1,
                         ss0, ss1)

    plsc.subcore_barrier()
    pltpu.sync_copy(acc.at[pl.ds(s * rpt, rpt)],
                    out_hbm.at[c, pl.ds(s * rpt, rpt)])


@functools.lru_cache(maxsize=None)
def _make_sc_msg(nchunk, t0):
    mesh = plsc.VectorSubcoreMesh(core_axis_name="c", subcore_axis_name="s")
    return pl.kernel(
        functools.partial(_sc_msg_body, nchunk, t0),
        out_type=jax.ShapeDtypeStruct((NC, NPAD, DIM), jnp.float32),
        mesh=mesh,
        scratch_types=[
            pltpu.VMEM_SHARED((NPAD, DIM), jnp.float32),
            pltpu.VMEM((SUP, CH), jnp.int32),
            pltpu.VMEM((SUP, CH), jnp.int32),
            pltpu.VMEM((CH, DIM), jnp.float32),
            pltpu.VMEM((CH, DIM), jnp.float32),
            pltpu.SemaphoreType.DMA,
            pltpu.SemaphoreType.DMA,
            pltpu.SemaphoreType.DMA,
            pltpu.SemaphoreType.DMA,
        ],
    )


def _sc_cnt_body(nchunk, ai_hbm, bi_hbm, coli_hbm, zeros_hbm,
                 out_hbm, acc, aba, bba, cba, rows0, rows1, ss0, ss1):
    c = lax.axis_index("c")
    s = lax.axis_index("s")
    wid = s * NC + c
    rpt = NPAD // NS
    pltpu.sync_copy(zeros_hbm.at[pl.ds(s * rpt, rpt)],
                    acc.at[pl.ds(s * rpt, rpt)])
    pltpu.sync_copy(zeros_hbm.at[pl.ds(0, CH)], rows0)
    pltpu.sync_copy(zeros_hbm.at[pl.ds(0, CH)], rows1)
    plsc.subcore_barrier()

    ones16 = jnp.full((16,), 1.0, jnp.float32)
    zeros16 = jnp.zeros((16,), jnp.float32)
    lane = lax.iota(jnp.int32, 16)
    rows_bufs = (rows0, rows1)
    sems = (ss0, ss1)

    def set_vals(j, rows, val):
        for g in range(CH // 16):
            kvec = lane + (g * 16)
            sl = pl.ds(g * 16, 16)
            plsc.store_scatter(rows, [kvec, aba[j, sl]], val)
            plsc.store_scatter(rows, [kvec, bba[j, sl] + 8], val)

    def scat(j, rows, sem):
        pltpu.async_copy(rows, acc.at[cba.at[j]], sem, add=True)

    def scat_wait(j, rows, sem):
        pltpu.make_async_copy(rows, acc.at[cba.at[j]], sem).wait()

    @pl.loop(0, nchunk // SUP)
    def _(t):
        base = wid * nchunk + t * SUP
        pltpu.sync_copy(ai_hbm.at[pl.ds(base, SUP)], aba)
        pltpu.sync_copy(bi_hbm.at[pl.ds(base, SUP)], bba)
        pltpu.sync_copy(coli_hbm.at[pl.ds(base, SUP)], cba)
        for j in range(SUP):
            sl = j % 2
            set_vals(j, rows_bufs[sl], ones16)
            scat(j, rows_bufs[sl], sems[sl])
            if j >= 1:
                psl = (j - 1) % 2
                scat_wait(j - 1, rows_bufs[psl], sems[psl])
                set_vals(j - 1, rows_bufs[psl], zeros16)
        scat_wait(SUP - 1, rows_bufs[(SUP - 1) % 2], sems[(SUP - 1) % 2])
        set_vals(SUP - 1, rows_bufs[(SUP - 1) % 2], zeros16)

    plsc.subcore_barrier()
    pltpu.sync_copy(acc.at[pl.ds(s * rpt, rpt)],
                    out_hbm.at[c, pl.ds(s * rpt, rpt)])


@functools.lru_cache(maxsize=None)
def _make_sc_cnt(nchunk):
    mesh = plsc.VectorSubcoreMesh(core_axis_name="c", subcore_axis_name="s")
    return pl.kernel(
        functools.partial(_sc_cnt_body, nchunk),
        out_type=jax.ShapeDtypeStruct((NC, NPAD, DIM), jnp.float32),
        mesh=mesh,
        scratch_types=[
            pltpu.VMEM_SHARED((NPAD, DIM), jnp.float32),
            pltpu.VMEM((SUP, CH), jnp.int32),
            pltpu.VMEM((SUP, CH), jnp.int32),
            pltpu.VMEM((SUP, CH), jnp.int32),
            pltpu.VMEM((CH, DIM), jnp.float32),
            pltpu.VMEM((CH, DIM), jnp.float32),
            pltpu.SemaphoreType.DMA,
            pltpu.SemaphoreType.DMA,
        ],
        compiler_params=pltpu.CompilerParams(needs_layout_passes=False),
    )


def _prep_body(x0_ref, x1_ref, e1_ref, e2_ref, w0_ref, hw0_ref):
    oh0 = (x0_ref[:] == lax.broadcasted_iota(jnp.int32, (NPAD, 8), 1)
           ).astype(jnp.float32)
    oh1 = (x1_ref[:] == lax.broadcasted_iota(jnp.int32, (NPAD, 8), 1)
           ).astype(jnp.float32)
    h0 = (jnp.dot(oh0, e1_ref[:], preferred_element_type=jnp.float32)
          + jnp.dot(oh1, e2_ref[:], preferred_element_type=jnp.float32))
    hw0_ref[:] = jnp.dot(h0, w0_ref[:], preferred_element_type=jnp.float32)


def _layer_body(nreal, relu, has_w, p_ref, hw_ref, cntp_ref, v_ref, g_ref,
                b_ref, wn_ref, out_ref):
    cnt = cntp_ref[0] + cntp_ref[1]
    scol = jnp.sum(cnt * v_ref[:], axis=1, keepdims=True)
    out = p_ref[0] + p_ref[1] + hw_ref[:] + scol
    rmask = lax.broadcasted_iota(jnp.int32, (NPAD, 1), 0) < nreal
    outm = jnp.where(rmask, out, 0.0)
    mean = jnp.sum(outm, axis=0, keepdims=True) * (1.0 / nreal)
    d = jnp.where(rmask, out - mean, 0.0)
    var = jnp.sum(d * d, axis=0, keepdims=True) * (1.0 / nreal)
    xn = (out - mean) * lax.rsqrt(var + 1e-5) * g_ref[:] + b_ref[:]
    if relu:
        xn = jnp.maximum(xn, 0.0)
    if has_w:
        out_ref[:] = jnp.dot(xn, wn_ref[:], preferred_element_type=jnp.float32)
    else:
        out_ref[:] = xn


def _pool_body(ngroups, h_ref, bt_ref, flw_ref, flb_ref, p1w_ref, p1b_ref,
               p2w_ref, p2b_ref, hg_ref, pred_ref):
    oht = (bt_ref[:] == lax.broadcasted_iota(jnp.int32, (ngroups, NPAD), 0)
           ).astype(jnp.float32)
    pooled = jnp.dot(oht, h_ref[:], preferred_element_type=jnp.float32)
    cnts = jnp.sum(oht, axis=1, keepdims=True)
    pooled = pooled / jnp.maximum(cnts, 1.0)
    hg = jnp.dot(pooled, flw_ref[:], preferred_element_type=jnp.float32) \
        + flb_ref[:]
    hg_ref[:] = hg
    t = jnp.dot(hg, p1w_ref[:], preferred_element_type=jnp.float32) \
        + p1b_ref[:]
    sp = jnp.maximum(t, 0.0) + jnp.log(1.0 + jnp.exp(-jnp.abs(t)))
    pred_ref[:] = jnp.dot(sp, p2w_ref[:], preferred_element_type=jnp.float32) \
        + p2b_ref[:]


def kernel(x, edge_index, edge_attr, batch, x_emb1, x_emb2, Ws, bs, ee1, ee2,
           gammas, betas, flW, flb, p1W, p1b, p2W, p2b):
    n = x.shape[0]
    e = edge_index.shape[1]
    nlayers = Ws.shape[0]
    feat = flW.shape[1]
    ngroups = 64

    nchunk = -(-e // (NW * CH))
    nchunk = -(-nchunk // SUP) * SUP
    epw = nchunk * CH
    e_pad = NW * epw

    f32 = jnp.float32
    i32 = jnp.int32

    x0 = jnp.pad(x[:, 0:1], ((0, NPAD - n), (0, 0))).astype(i32)
    x1 = jnp.pad(x[:, 1:2], ((0, NPAD - n), (0, 0))).astype(i32)
    e1 = x_emb1[:8].astype(f32)
    e2 = jnp.pad(x_emb2, ((0, 5), (0, 0))).astype(f32)
    rowi = jnp.pad(edge_index[0], (0, e_pad - e),
                   constant_values=NPAD - 1).astype(i32).reshape(-1, CH)
    coli = jnp.pad(edge_index[1], (0, e_pad - e),
                   constant_values=NPAD - 1).astype(i32).reshape(-1, CH)
    ai = jnp.pad(edge_attr[:, 0], (0, e_pad - e),
                 constant_values=3).astype(i32).reshape(-1, CH)
    bi = jnp.pad(edge_attr[:, 1], (0, e_pad - e),
                 constant_values=3).astype(i32).reshape(-1, CH)
    bt = jnp.pad(batch, (0, NPAD - n),
                 constant_values=ngroups).reshape(1, NPAD).astype(i32)
    zeros_nk = jnp.zeros((NPAD, DIM), f32)
    p2w_pad = jnp.pad(p2W, ((0, 0), (0, DIM - p2W.shape[1]))).astype(f32)
    p2b_pad = jnp.pad(p2b, (0, DIM - p2b.shape[0])).reshape(1, DIM).astype(f32)

    cntp = _make_sc_cnt(nchunk)(ai, bi, coli, zeros_nk)

    prep = pl.pallas_call(
        _prep_body,
        out_shape=jax.ShapeDtypeStruct((NPAD, DIM), f32),
    )
    hw = prep(x0, x1, e1, e2, Ws[0].astype(f32))

    sc_msg = _make_sc_msg(nchunk, 16)
    for l in range(nlayers):
        partials = sc_msg(hw, rowi, coli, zeros_nk)
        has_w = l < nlayers - 1
        v = jnp.concatenate([
            ee1[l, :3, 0], jnp.zeros((5,), f32),
            ee2[l, :3, 0], jnp.zeros((DIM - 11,), f32)]).reshape(1, DIM)
        g = gammas[l].reshape(1, DIM).astype(f32)
        b = betas[l].reshape(1, DIM).astype(f32)
        wn = (Ws[l + 1] if has_w else jnp.zeros((DIM, DIM))).astype(f32)
        layer = pl.pallas_call(
            functools.partial(_layer_body, n, l < nlayers - 1, has_w),
            out_shape=jax.ShapeDtypeStruct((NPAD, DIM), f32),
        )
        hw = layer(partials, hw, cntp, v, g, b, wn)
    h_final = hw

    pool = pl.pallas_call(
        functools.partial(_pool_body, ngroups),
        out_shape=(
            jax.ShapeDtypeStruct((ngroups, feat), f32),
            jax.ShapeDtypeStruct((ngroups, DIM), f32),
        ),
    )
    hg, pred_pad = pool(h_final, bt, flW.astype(f32),
                        flb.reshape(1, feat).astype(f32), p1W.astype(f32),
                        p1b.reshape(1, p1W.shape[1]).astype(f32), p2w_pad,
                        p2b_pad)
    return (hg, pred_pad[:, :p2W.shape[1]])
